# Initial kernel scaffold; baseline (speedup 1.0000x reference)
import jax
import jax.numpy as jnp
from jax.experimental import pallas as pl


def kernel(x, params):
    raise NotImplementedError("write your pallas kernel here")


# speedup vs baseline: 2.5730x; 2.5730x over previous
import math

import jax
import jax.numpy as jnp
from jax import lax
from jax.experimental import pallas as pl
from jax.experimental.pallas import tpu as pltpu

F32 = jnp.float32
BF16 = jnp.bfloat16

B = 8
S = 514
T = 528
ED = 768
EH = 12
DD = 512
DH = 8
BIN = 100
PAD_ID = 103.0
MASK_ID = 102.0
NEG = -1e9


def _prep_kernel(xb_ref, labels_ref, rank_ref, num_ref, maskadd_ref, encval_ref):
    xb = xb_ref[...]
    l = (xb > 0.0).astype(F32)
    ii = lax.broadcasted_iota(jnp.int32, (T, T), 0).astype(F32)
    jj = lax.broadcasted_iota(jnp.int32, (T, T), 1).astype(F32)
    lower = (ii <= jj).astype(F32)
    csum = jnp.dot(l, lower, preferred_element_type=F32)
    rank = csum - 1.0
    num = csum[:, T - 1:T]
    jrow = lax.broadcasted_iota(jnp.int32, (1, T), 1).astype(F32)
    labels_ref[...] = l
    rank_ref[...] = rank
    num_ref[...] = num
    maskadd_ref[...] = jnp.where(jrow >= num, NEG, 0.0)
    jcol = lax.broadcasted_iota(jnp.int32, (T, T), 0).astype(F32)
    for b in range(B):
        rb = rank[b:b + 1, :]
        lb = l[b:b + 1, :]
        mb = jnp.where((jcol == rb) & (lb > 0.0), 1.0, 0.0)
        val = lax.dot_general(xb[b:b + 1, :], mb,
                              (((1,), (1,)), ((), ())),
                              preferred_element_type=F32)
        encval_ref[b:b + 1, :] = jnp.where(jrow >= num[b:b + 1, :], PAD_ID, val)


def _prep(xb_pad):
    return pl.pallas_call(
        _prep_kernel,
        out_shape=[
            jax.ShapeDtypeStruct((B, T), F32),
            jax.ShapeDtypeStruct((B, T), F32),
            jax.ShapeDtypeStruct((B, 1), F32),
            jax.ShapeDtypeStruct((B, T), F32),
            jax.ShapeDtypeStruct((B, T), F32),
        ],
    )(xb_pad)


def _embed_kernel(v_ref, w1_ref, b1_ref, w2t_ref, b2_ref, emb_ref, em_ref,
                  ep_ref, out_ref):
    v = v_ref[...]
    h = v * w1_ref[...] + b1_ref[...]
    h = jnp.where(h >= 0.0, h, 0.1 * h)
    h2 = jnp.dot(h, w2t_ref[...], preferred_element_type=F32) + b2_ref[...]
    logits = h + h2
    logits = logits - jnp.max(logits, axis=1, keepdims=True)
    w = jnp.exp(logits)
    w = w / jnp.sum(w, axis=1, keepdims=True)
    e = jnp.dot(w.astype(BF16), emb_ref[...], preferred_element_type=F32)
    e = jnp.where(v == MASK_ID, em_ref[...], e)
    e = jnp.where(v == PAD_ID, ep_ref[...], e)
    out_ref[...] = e


def _embed(tokens, p):
    n = tokens.shape[0]
    bm = 528
    grid = (n // bm,)
    return pl.pallas_call(
        _embed_kernel,
        grid=grid,
        in_specs=[
            pl.BlockSpec((bm, 1), lambda i: (i, 0)),
            pl.BlockSpec((1, BIN), lambda i: (0, 0)),
            pl.BlockSpec((1, BIN), lambda i: (0, 0)),
            pl.BlockSpec((BIN, BIN), lambda i: (0, 0)),
            pl.BlockSpec((1, BIN), lambda i: (0, 0)),
            pl.BlockSpec((BIN, ED), lambda i: (0, 0)),
            pl.BlockSpec((1, ED), lambda i: (0, 0)),
            pl.BlockSpec((1, ED), lambda i: (0, 0)),
        ],
        out_specs=pl.BlockSpec((bm, ED), lambda i: (i, 0)),
        out_shape=jax.ShapeDtypeStruct((n, ED), F32),
    )(tokens,
      p["te_w1"].reshape(1, BIN),
      p["te_b1"].reshape(1, BIN),
      p["te_w2"].T,
      p["te_b2"].reshape(1, BIN),
      p["te_emb"].astype(BF16),
      p["te_emb_mask"].reshape(1, ED),
      p["te_emb_pad"].reshape(1, ED))


def _posgather_kernel(l_ref, r_ref, num_ref, h0e_ref, pe_ref, plast_ref,
                      out_ref):
    lb = l_ref[0]
    rb = r_ref[0]
    jcol = lax.broadcasted_iota(jnp.int32, (T, T), 0).astype(F32)
    mb = jnp.where((jcol == rb) & (lb > 0.0), 1.0, 0.0)
    pe = jnp.dot(mb.astype(BF16), pe_ref[...], preferred_element_type=F32)
    padcol = (lax.broadcasted_iota(jnp.int32, (T, 1), 0).astype(F32) >= num_ref[0]).astype(F32)
    pe = pe + padcol * plast_ref[...]
    out_ref[0] = h0e_ref[0] + pe


def _posgather(labels, rank, num, h0e, pos_pad, pos_last):
    return pl.pallas_call(
        _posgather_kernel,
        grid=(B,),
        in_specs=[
            pl.BlockSpec((1, 1, T), lambda b: (b, 0, 0)),
            pl.BlockSpec((1, 1, T), lambda b: (b, 0, 0)),
            pl.BlockSpec((1, 1, 1), lambda b: (b, 0, 0)),
            pl.BlockSpec((1, T, ED), lambda b: (b, 0, 0)),
            pl.BlockSpec((T, ED), lambda b: (0, 0)),
            pl.BlockSpec((1, ED), lambda b: (0, 0)),
        ],
        out_specs=pl.BlockSpec((1, T, ED), lambda b: (b, 0, 0)),
        out_shape=jax.ShapeDtypeStruct((B, T, ED), F32),
    )(labels.reshape(B, 1, T), rank.reshape(B, 1, T), num.reshape(B, 1, 1),
      h0e, pos_pad, pos_last)


def _attn_kernel(x_ref, wqkv_ref, bqkv_ref, wo_ref, bo_ref, g_ref, bb_ref,
                 mask_ref, out_ref, acc_ref, *, heads, dim):
    x = x_ref[0]
    qkv = jnp.dot(x.astype(BF16), wqkv_ref[...],
                  preferred_element_type=F32) + bqkv_ref[...]
    dh = dim // heads
    scale = 1.0 / math.sqrt(dh)
    mask = mask_ref[0]
    for h in range(heads):
        q = qkv[:, h * dh:(h + 1) * dh]
        k = qkv[:, dim + h * dh:dim + (h + 1) * dh]
        v = qkv[:, 2 * dim + h * dh:2 * dim + (h + 1) * dh]
        s = lax.dot_general(q.astype(BF16), k.astype(BF16),
                            (((1,), (1,)), ((), ())),
                            preferred_element_type=F32) * scale + mask
        s = s - jnp.max(s, axis=1, keepdims=True)
        ps = jnp.exp(s)
        ps = ps / jnp.sum(ps, axis=1, keepdims=True)
        acc_ref[:, h * dh:(h + 1) * dh] = jnp.dot(
            ps.astype(BF16), v.astype(BF16), preferred_element_type=F32)
    o = jnp.dot(acc_ref[...].astype(BF16), wo_ref[...],
                preferred_element_type=F32) + bo_ref[...]
    y = x + o
    m = jnp.mean(y, axis=1, keepdims=True)
    d = y - m
    var = jnp.mean(d * d, axis=1, keepdims=True)
    out_ref[0] = d * lax.rsqrt(var + 1e-5) * g_ref[...] + bb_ref[...]


def _attn_block(x, lp, mask, heads, dim):
    import functools
    return pl.pallas_call(
        functools.partial(_attn_kernel, heads=heads, dim=dim),
        grid=(B,),
        in_specs=[
            pl.BlockSpec((1, T, dim), lambda b: (b, 0, 0)),
            pl.BlockSpec((dim, 3 * dim), lambda b: (0, 0)),
            pl.BlockSpec((1, 3 * dim), lambda b: (0, 0)),
            pl.BlockSpec((dim, dim), lambda b: (0, 0)),
            pl.BlockSpec((1, dim), lambda b: (0, 0)),
            pl.BlockSpec((1, dim), lambda b: (0, 0)),
            pl.BlockSpec((1, dim), lambda b: (0, 0)),
            pl.BlockSpec((1, 1, T), lambda b: (b, 0, 0)),
        ],
        out_specs=pl.BlockSpec((1, T, dim), lambda b: (b, 0, 0)),
        out_shape=jax.ShapeDtypeStruct((B, T, dim), F32),
        scratch_shapes=[pltpu.VMEM((T, dim), F32)],
    )(x, lp["WqkvT"], lp["bqkv"], lp["WoT"], lp["bo"], lp["ln1_g"],
      lp["ln1_b"], mask.reshape(B, 1, T))


def _mlp_kernel(x_ref, w1_ref, b1_ref, w2_ref, b2_ref, g_ref, bb_ref, out_ref):
    x = x_ref[0]
    h = jnp.dot(x.astype(BF16), w1_ref[...],
                preferred_element_type=F32) + b1_ref[...]
    h = h * 0.5 * (1.0 + lax.erf(h * (1.0 / math.sqrt(2.0))))
    o = jnp.dot(h.astype(BF16), w2_ref[...],
                preferred_element_type=F32) + b2_ref[...]
    y = x + o
    m = jnp.mean(y, axis=1, keepdims=True)
    d = y - m
    var = jnp.mean(d * d, axis=1, keepdims=True)
    out_ref[0] = d * lax.rsqrt(var + 1e-5) * g_ref[...] + bb_ref[...]


def _mlp_block(x, lp, dim, ff):
    return pl.pallas_call(
        _mlp_kernel,
        grid=(B,),
        in_specs=[
            pl.BlockSpec((1, T, dim), lambda b: (b, 0, 0)),
            pl.BlockSpec((dim, ff), lambda b: (0, 0)),
            pl.BlockSpec((1, ff), lambda b: (0, 0)),
            pl.BlockSpec((ff, dim), lambda b: (0, 0)),
            pl.BlockSpec((1, dim), lambda b: (0, 0)),
            pl.BlockSpec((1, dim), lambda b: (0, 0)),
            pl.BlockSpec((1, dim), lambda b: (0, 0)),
        ],
        out_specs=pl.BlockSpec((1, T, dim), lambda b: (b, 0, 0)),
        out_shape=jax.ShapeDtypeStruct((B, T, dim), F32),
    )(x, lp["W1T"], lp["b1"], lp["W2T"], lp["b2"], lp["ln2_g"], lp["ln2_b"])


def _ln_kernel(x_ref, g_ref, bb_ref, out_ref):
    x = x_ref[0]
    m = jnp.mean(x, axis=1, keepdims=True)
    d = x - m
    var = jnp.mean(d * d, axis=1, keepdims=True)
    out_ref[0] = d * lax.rsqrt(var + 1e-5) * g_ref[...] + bb_ref[...]


def _ln(x, g, bb, dim):
    return pl.pallas_call(
        _ln_kernel,
        grid=(B,),
        in_specs=[
            pl.BlockSpec((1, T, dim), lambda b: (b, 0, 0)),
            pl.BlockSpec((1, dim), lambda b: (0, 0)),
            pl.BlockSpec((1, dim), lambda b: (0, 0)),
        ],
        out_specs=pl.BlockSpec((1, T, dim), lambda b: (b, 0, 0)),
        out_shape=jax.ShapeDtypeStruct((B, T, dim), F32),
    )(x, g.reshape(1, dim), bb.reshape(1, dim))


def _scatter_kernel(l_ref, r_ref, henc_ref, e_ref, pos_ref, wdec_ref,
                    bdec_ref, out_ref):
    lb = l_ref[0]
    rb = r_ref[0]
    jcol = lax.broadcasted_iota(jnp.int32, (T, T), 0).astype(F32)
    mb = jnp.where((jcol == rb) & (lb > 0.0), 1.0, 0.0)
    gathered = lax.dot_general(mb.astype(BF16), henc_ref[0].astype(BF16),
                               (((0,), (0,)), ((), ())),
                               preferred_element_type=F32)
    ones = jnp.ones((T, 1), F32)
    lcol = lax.dot_general(mb, ones, (((0,), (0,)), ((), ())),
                           preferred_element_type=F32)
    base = gathered + (1.0 - lcol) * e_ref[0] + pos_ref[...]
    out_ref[0] = jnp.dot(base.astype(BF16), wdec_ref[...],
                         preferred_element_type=F32) + bdec_ref[...]


def _scatterback(labels, rank, h_enc, e_full, dec_pos_pad, wdec_t, bdec):
    return pl.pallas_call(
        _scatter_kernel,
        grid=(B,),
        in_specs=[
            pl.BlockSpec((1, 1, T), lambda b: (b, 0, 0)),
            pl.BlockSpec((1, 1, T), lambda b: (b, 0, 0)),
            pl.BlockSpec((1, T, ED), lambda b: (b, 0, 0)),
            pl.BlockSpec((1, T, ED), lambda b: (b, 0, 0)),
            pl.BlockSpec((T, ED), lambda b: (0, 0)),
            pl.BlockSpec((ED, DD), lambda b: (0, 0)),
            pl.BlockSpec((1, DD), lambda b: (0, 0)),
        ],
        out_specs=pl.BlockSpec((1, T, DD), lambda b: (b, 0, 0)),
        out_shape=jax.ShapeDtypeStruct((B, T, DD), F32),
    )(labels.reshape(B, 1, T), rank.reshape(B, 1, T), h_enc, e_full,
      dec_pos_pad, wdec_t, bdec)


def _prep_layer(p, dim):
    return {
        "WqkvT": p["Wqkv"].T.astype(BF16),
        "bqkv": p["bqkv"].reshape(1, 3 * dim),
        "WoT": p["Wo"].T.astype(BF16),
        "bo": p["bo"].reshape(1, dim),
        "ln1_g": p["ln1_g"].reshape(1, dim),
        "ln1_b": p["ln1_b"].reshape(1, dim),
        "W1T": p["W1"].T.astype(BF16),
        "b1": p["b1"].reshape(1, 4 * dim),
        "W2T": p["W2"].T.astype(BF16),
        "b2": p["b2"].reshape(1, dim),
        "ln2_g": p["ln2_g"].reshape(1, dim),
        "ln2_b": p["ln2_b"].reshape(1, dim),
    }


def kernel(x, params):
    genes = x[:, :-1]
    li = jnp.log10(x[:, -1:])
    xb = jnp.concatenate([genes, li, li], axis=1)
    xb_pad = jnp.pad(xb, ((0, 0), (0, T - S)))

    labels, rank, num, maskadd_enc, enc_val = _prep(xb_pad)

    tokens = jnp.concatenate(
        [enc_val.reshape(-1, 1), xb_pad.reshape(-1, 1)], axis=0)
    e_all = _embed(tokens, params)
    h0e = e_all[:B * T].reshape(B, T, ED)
    e_full = e_all[B * T:].reshape(B, T, ED)

    pos_pad = jnp.pad(params["pos_emb"][:S], ((0, T - S), (0, 0))).astype(BF16)
    pos_last = params["pos_emb"][S:S + 1]
    h = _posgather(labels, rank, num, h0e, pos_pad, pos_last)

    enc_layers = [_prep_layer(p, ED) for p in params["enc_layers"]]
    for lp in enc_layers:
        h = _attn_block(h, lp, maskadd_enc, EH, ED)
        h = _mlp_block(h, lp, ED, 4 * ED)
    h = _ln(h, params["enc_norm_g"], params["enc_norm_b"], ED)

    dec_pos_pad = jnp.pad(params["dec_pos_emb"][:S], ((0, T - S), (0, 0)))
    d = _scatterback(labels, rank, h, e_full, dec_pos_pad,
                     params["dec_embed_W"].T.astype(BF16),
                     params["dec_embed_b"].reshape(1, DD))

    maskadd_dec = jnp.broadcast_to(
        jnp.where(jnp.arange(T, dtype=F32) >= S, NEG, 0.0)[None, :], (B, T))

    dec_layers = [_prep_layer(p, DD) for p in params["dec_layers"]]
    for lp in dec_layers:
        d = _attn_block(d, lp, maskadd_dec, DH, DD)
        d = _mlp_block(d, lp, DD, 4 * DD)
    d = _ln(d, params["dec_norm_g"], params["dec_norm_b"], DD)
    return d[:, :S, :]


# speedup vs baseline: 2.7101x; 1.0533x over previous
import math

import jax
import jax.numpy as jnp
from jax import lax
from jax.experimental import pallas as pl
from jax.experimental.pallas import tpu as pltpu

F32 = jnp.float32
BF16 = jnp.bfloat16

B = 8
S = 514
T = 528
ED = 768
EH = 12
DD = 512
DH = 8
BIN = 100
PAD_ID = 103.0
MASK_ID = 102.0
NEG = -1e9


def _prep_kernel(xb_ref, labels_ref, rank_ref, num_ref, maskadd_ref, encval_ref):
    xb = xb_ref[...]
    l = (xb > 0.0).astype(F32)
    ii = lax.broadcasted_iota(jnp.int32, (T, T), 0).astype(F32)
    jj = lax.broadcasted_iota(jnp.int32, (T, T), 1).astype(F32)
    lower = (ii <= jj).astype(F32)
    csum = jnp.dot(l, lower, preferred_element_type=F32)
    rank = csum - 1.0
    num = csum[:, T - 1:T]
    jrow = lax.broadcasted_iota(jnp.int32, (1, T), 1).astype(F32)
    labels_ref[...] = l
    rank_ref[...] = rank
    num_ref[...] = num
    maskadd_ref[...] = jnp.where(jrow >= num, NEG, 0.0)
    jcol = lax.broadcasted_iota(jnp.int32, (T, T), 0).astype(F32)
    for b in range(B):
        rb = rank[b:b + 1, :]
        lb = l[b:b + 1, :]
        mb = jnp.where((jcol == rb) & (lb > 0.0), 1.0, 0.0)
        val = lax.dot_general(xb[b:b + 1, :], mb,
                              (((1,), (1,)), ((), ())),
                              preferred_element_type=F32)
        encval_ref[b:b + 1, :] = jnp.where(jrow >= num[b:b + 1, :], PAD_ID, val)


def _prep(xb_pad):
    return pl.pallas_call(
        _prep_kernel,
        out_shape=[
            jax.ShapeDtypeStruct((B, T), F32),
            jax.ShapeDtypeStruct((B, T), F32),
            jax.ShapeDtypeStruct((B, 1), F32),
            jax.ShapeDtypeStruct((B, T), F32),
            jax.ShapeDtypeStruct((B, T), F32),
        ],
    )(xb_pad)


def _embed_kernel(v_ref, w1_ref, b1_ref, w2t_ref, b2_ref, emb_ref, em_ref,
                  ep_ref, out_ref):
    v = v_ref[...]
    h = v * w1_ref[...] + b1_ref[...]
    h = jnp.where(h >= 0.0, h, 0.1 * h)
    h2 = jnp.dot(h, w2t_ref[...], preferred_element_type=F32) + b2_ref[...]
    logits = h + h2
    logits = logits - jnp.max(logits, axis=1, keepdims=True)
    w = jnp.exp(logits)
    w = w / jnp.sum(w, axis=1, keepdims=True)
    e = jnp.dot(w.astype(BF16), emb_ref[...], preferred_element_type=F32)
    e = jnp.where(v == MASK_ID, em_ref[...], e)
    e = jnp.where(v == PAD_ID, ep_ref[...], e)
    out_ref[...] = e


def _embed(tokens, p):
    n = tokens.shape[0]
    bm = 528
    grid = (n // bm,)
    return pl.pallas_call(
        _embed_kernel,
        grid=grid,
        in_specs=[
            pl.BlockSpec((bm, 1), lambda i: (i, 0)),
            pl.BlockSpec((1, BIN), lambda i: (0, 0)),
            pl.BlockSpec((1, BIN), lambda i: (0, 0)),
            pl.BlockSpec((BIN, BIN), lambda i: (0, 0)),
            pl.BlockSpec((1, BIN), lambda i: (0, 0)),
            pl.BlockSpec((BIN, ED), lambda i: (0, 0)),
            pl.BlockSpec((1, ED), lambda i: (0, 0)),
            pl.BlockSpec((1, ED), lambda i: (0, 0)),
        ],
        out_specs=pl.BlockSpec((bm, ED), lambda i: (i, 0)),
        out_shape=jax.ShapeDtypeStruct((n, ED), F32),
    )(tokens,
      p["te_w1"].reshape(1, BIN),
      p["te_b1"].reshape(1, BIN),
      p["te_w2"].T,
      p["te_b2"].reshape(1, BIN),
      p["te_emb"].astype(BF16),
      p["te_emb_mask"].reshape(1, ED),
      p["te_emb_pad"].reshape(1, ED))


def _posgather_kernel(l_ref, r_ref, num_ref, h0e_ref, pe_ref, plast_ref,
                      out_ref):
    lb = l_ref[0]
    rb = r_ref[0]
    jcol = lax.broadcasted_iota(jnp.int32, (T, T), 0).astype(F32)
    mb = jnp.where((jcol == rb) & (lb > 0.0), 1.0, 0.0)
    pe = jnp.dot(mb.astype(BF16), pe_ref[...], preferred_element_type=F32)
    padcol = (lax.broadcasted_iota(jnp.int32, (T, 1), 0).astype(F32) >= num_ref[0]).astype(F32)
    pe = pe + padcol * plast_ref[...]
    out_ref[0] = h0e_ref[0] + pe


def _posgather(labels, rank, num, h0e, pos_pad, pos_last):
    return pl.pallas_call(
        _posgather_kernel,
        grid=(B,),
        in_specs=[
            pl.BlockSpec((1, 1, T), lambda b: (b, 0, 0)),
            pl.BlockSpec((1, 1, T), lambda b: (b, 0, 0)),
            pl.BlockSpec((1, 1, 1), lambda b: (b, 0, 0)),
            pl.BlockSpec((1, T, ED), lambda b: (b, 0, 0)),
            pl.BlockSpec((T, ED), lambda b: (0, 0)),
            pl.BlockSpec((1, ED), lambda b: (0, 0)),
        ],
        out_specs=pl.BlockSpec((1, T, ED), lambda b: (b, 0, 0)),
        out_shape=jax.ShapeDtypeStruct((B, T, ED), F32),
    )(labels.reshape(B, 1, T), rank.reshape(B, 1, T), num.reshape(B, 1, 1),
      h0e, pos_pad, pos_last)


def _stack_kernel(x_ref, wqkv_ref, bqkv_ref, wo_ref, bo_ref, g1_ref, bb1_ref,
                  w1_ref, b1_ref, w2_ref, b2_ref, g2_ref, bb2_ref, mask_ref,
                  out_ref, acc_ref, *, heads, dim, ff):
    x = x_ref[0]
    qkv = lax.dot_general(x.astype(BF16), wqkv_ref[0],
                          (((1,), (1,)), ((), ())),
                          preferred_element_type=F32) + bqkv_ref[0]
    dh = dim // heads
    scale = 1.0 / math.sqrt(dh)
    mask = mask_ref[0]
    for h in range(heads):
        q = qkv[:, h * dh:(h + 1) * dh]
        k = qkv[:, dim + h * dh:dim + (h + 1) * dh]
        v = qkv[:, 2 * dim + h * dh:2 * dim + (h + 1) * dh]
        s = lax.dot_general(q.astype(BF16), k.astype(BF16),
                            (((1,), (1,)), ((), ())),
                            preferred_element_type=F32) * scale + mask
        s = s - jnp.max(s, axis=1, keepdims=True)
        ps = jnp.exp(s)
        ps = ps / jnp.sum(ps, axis=1, keepdims=True)
        acc_ref[:, h * dh:(h + 1) * dh] = jnp.dot(
            ps.astype(BF16), v.astype(BF16), preferred_element_type=F32)
    o = lax.dot_general(acc_ref[...].astype(BF16), wo_ref[0],
                        (((1,), (1,)), ((), ())),
                        preferred_element_type=F32) + bo_ref[0]
    y = x + o
    m = jnp.mean(y, axis=1, keepdims=True)
    d = y - m
    var = jnp.mean(d * d, axis=1, keepdims=True)
    y = d * lax.rsqrt(var + 1e-5) * g1_ref[0] + bb1_ref[0]

    hh = lax.dot_general(y.astype(BF16), w1_ref[0],
                         (((1,), (1,)), ((), ())),
                         preferred_element_type=F32) + b1_ref[0]
    hh = hh * 0.5 * (1.0 + lax.erf(hh * (1.0 / math.sqrt(2.0))))
    o2 = lax.dot_general(hh.astype(BF16), w2_ref[0],
                         (((1,), (1,)), ((), ())),
                         preferred_element_type=F32) + b2_ref[0]
    y2 = y + o2
    m2 = jnp.mean(y2, axis=1, keepdims=True)
    d2 = y2 - m2
    var2 = jnp.mean(d2 * d2, axis=1, keepdims=True)
    out_ref[0] = d2 * lax.rsqrt(var2 + 1e-5) * g2_ref[0] + bb2_ref[0]


def _stack(x, layers, mask, heads, dim, ff):
    import functools
    depth = len(layers)
    wqkv = jnp.stack([p["Wqkv"] for p in layers]).astype(BF16)
    bqkv = jnp.stack([p["bqkv"].reshape(1, 3 * dim) for p in layers])
    wo = jnp.stack([p["Wo"] for p in layers]).astype(BF16)
    bo = jnp.stack([p["bo"].reshape(1, dim) for p in layers])
    g1 = jnp.stack([p["ln1_g"].reshape(1, dim) for p in layers])
    bb1 = jnp.stack([p["ln1_b"].reshape(1, dim) for p in layers])
    w1 = jnp.stack([p["W1"] for p in layers]).astype(BF16)
    b1 = jnp.stack([p["b1"].reshape(1, ff) for p in layers])
    w2 = jnp.stack([p["W2"] for p in layers]).astype(BF16)
    b2 = jnp.stack([p["b2"].reshape(1, dim) for p in layers])
    g2 = jnp.stack([p["ln2_g"].reshape(1, dim) for p in layers])
    bb2 = jnp.stack([p["ln2_b"].reshape(1, dim) for p in layers])
    lb = lambda l, b: (b, 0, 0)
    lw3 = lambda l, b: (l, 0, 0)
    return pl.pallas_call(
        functools.partial(_stack_kernel, heads=heads, dim=dim, ff=ff),
        grid=(depth, B),
        in_specs=[
            pl.BlockSpec((1, T, dim), lb),
            pl.BlockSpec((1, 3 * dim, dim), lw3),
            pl.BlockSpec((1, 1, 3 * dim), lw3),
            pl.BlockSpec((1, dim, dim), lw3),
            pl.BlockSpec((1, 1, dim), lw3),
            pl.BlockSpec((1, 1, dim), lw3),
            pl.BlockSpec((1, 1, dim), lw3),
            pl.BlockSpec((1, ff, dim), lw3),
            pl.BlockSpec((1, 1, ff), lw3),
            pl.BlockSpec((1, dim, ff), lw3),
            pl.BlockSpec((1, 1, dim), lw3),
            pl.BlockSpec((1, 1, dim), lw3),
            pl.BlockSpec((1, 1, dim), lw3),
            pl.BlockSpec((1, 1, T), lambda l, b: (b, 0, 0)),
        ],
        out_specs=pl.BlockSpec((1, T, dim), lb),
        out_shape=jax.ShapeDtypeStruct((B, T, dim), F32),
        scratch_shapes=[pltpu.VMEM((T, dim), F32)],
        input_output_aliases={0: 0},
    )(x, wqkv, bqkv, wo, bo, g1, bb1, w1, b1, w2, b2, g2, bb2,
      mask.reshape(B, 1, T))


def _ln_kernel(x_ref, g_ref, bb_ref, out_ref):
    x = x_ref[0]
    m = jnp.mean(x, axis=1, keepdims=True)
    d = x - m
    var = jnp.mean(d * d, axis=1, keepdims=True)
    out_ref[0] = d * lax.rsqrt(var + 1e-5) * g_ref[...] + bb_ref[...]


def _ln(x, g, bb, dim):
    return pl.pallas_call(
        _ln_kernel,
        grid=(B,),
        in_specs=[
            pl.BlockSpec((1, T, dim), lambda b: (b, 0, 0)),
            pl.BlockSpec((1, dim), lambda b: (0, 0)),
            pl.BlockSpec((1, dim), lambda b: (0, 0)),
        ],
        out_specs=pl.BlockSpec((1, T, dim), lambda b: (b, 0, 0)),
        out_shape=jax.ShapeDtypeStruct((B, T, dim), F32),
    )(x, g.reshape(1, dim), bb.reshape(1, dim))


def _scatter_kernel(l_ref, r_ref, henc_ref, e_ref, pos_ref, wdec_ref,
                    bdec_ref, out_ref):
    lb = l_ref[0]
    rb = r_ref[0]
    jcol = lax.broadcasted_iota(jnp.int32, (T, T), 0).astype(F32)
    mb = jnp.where((jcol == rb) & (lb > 0.0), 1.0, 0.0)
    gathered = lax.dot_general(mb.astype(BF16), henc_ref[0].astype(BF16),
                               (((0,), (0,)), ((), ())),
                               preferred_element_type=F32)
    ones = jnp.ones((T, 1), F32)
    lcol = lax.dot_general(mb, ones, (((0,), (0,)), ((), ())),
                           preferred_element_type=F32)
    base = gathered + (1.0 - lcol) * e_ref[0] + pos_ref[...]
    out_ref[0] = jnp.dot(base.astype(BF16), wdec_ref[...],
                         preferred_element_type=F32) + bdec_ref[...]


def _scatterback(labels, rank, h_enc, e_full, dec_pos_pad, wdec_t, bdec):
    return pl.pallas_call(
        _scatter_kernel,
        grid=(B,),
        in_specs=[
            pl.BlockSpec((1, 1, T), lambda b: (b, 0, 0)),
            pl.BlockSpec((1, 1, T), lambda b: (b, 0, 0)),
            pl.BlockSpec((1, T, ED), lambda b: (b, 0, 0)),
            pl.BlockSpec((1, T, ED), lambda b: (b, 0, 0)),
            pl.BlockSpec((T, ED), lambda b: (0, 0)),
            pl.BlockSpec((ED, DD), lambda b: (0, 0)),
            pl.BlockSpec((1, DD), lambda b: (0, 0)),
        ],
        out_specs=pl.BlockSpec((1, T, DD), lambda b: (b, 0, 0)),
        out_shape=jax.ShapeDtypeStruct((B, T, DD), F32),
    )(labels.reshape(B, 1, T), rank.reshape(B, 1, T), h_enc, e_full,
      dec_pos_pad, wdec_t, bdec)


def kernel(x, params):
    genes = x[:, :-1]
    li = jnp.log10(x[:, -1:])
    xb = jnp.concatenate([genes, li, li], axis=1)
    xb_pad = jnp.pad(xb, ((0, 0), (0, T - S)))

    labels, rank, num, maskadd_enc, enc_val = _prep(xb_pad)

    tokens = jnp.concatenate(
        [enc_val.reshape(-1, 1), xb_pad.reshape(-1, 1)], axis=0)
    e_all = _embed(tokens, params)
    h0e = e_all[:B * T].reshape(B, T, ED)
    e_full = e_all[B * T:].reshape(B, T, ED)

    pos_pad = jnp.pad(params["pos_emb"][:S], ((0, T - S), (0, 0))).astype(BF16)
    pos_last = params["pos_emb"][S:S + 1]
    h = _posgather(labels, rank, num, h0e, pos_pad, pos_last)

    h = _stack(h, params["enc_layers"], maskadd_enc, EH, ED, 4 * ED)
    h = _ln(h, params["enc_norm_g"], params["enc_norm_b"], ED)

    dec_pos_pad = jnp.pad(params["dec_pos_emb"][:S], ((0, T - S), (0, 0)))
    d = _scatterback(labels, rank, h, e_full, dec_pos_pad,
                     params["dec_embed_W"].T.astype(BF16),
                     params["dec_embed_b"].reshape(1, DD))

    maskadd_dec = jnp.broadcast_to(
        jnp.where(jnp.arange(T, dtype=F32) >= S, NEG, 0.0)[None, :], (B, T))

    d = _stack(d, params["dec_layers"], maskadd_dec, DH, DD, 4 * DD)
    d = _ln(d, params["dec_norm_g"], params["dec_norm_b"], DD)
    return d[:, :S, :]


# speedup vs baseline: 3.0933x; 1.1414x over previous
import math

import jax
import jax.numpy as jnp
from jax import lax
from jax.experimental import pallas as pl
from jax.experimental.pallas import tpu as pltpu

F32 = jnp.float32
BF16 = jnp.bfloat16

B = 8
S = 514
T = 528
ED = 768
EH = 12
DD = 512
DH = 8
BIN = 100
PAD_ID = 103.0
MASK_ID = 102.0
NEG = -1e9


def _prep_kernel(xb_ref, labels_ref, rank_ref, num_ref, maskadd_ref, encval_ref):
    xb = xb_ref[...]
    l = (xb > 0.0).astype(F32)
    ii = lax.broadcasted_iota(jnp.int32, (T, T), 0).astype(F32)
    jj = lax.broadcasted_iota(jnp.int32, (T, T), 1).astype(F32)
    lower = (ii <= jj).astype(F32)
    csum = jnp.dot(l, lower, preferred_element_type=F32)
    rank = csum - 1.0
    num = csum[:, T - 1:T]
    jrow = lax.broadcasted_iota(jnp.int32, (1, T), 1).astype(F32)
    labels_ref[...] = l
    rank_ref[...] = rank
    num_ref[...] = num
    maskadd_ref[...] = jnp.where(jrow >= num, NEG, 0.0)
    jcol = lax.broadcasted_iota(jnp.int32, (T, T), 0).astype(F32)
    for b in range(B):
        rb = rank[b:b + 1, :]
        lb = l[b:b + 1, :]
        mb = jnp.where((jcol == rb) & (lb > 0.0), 1.0, 0.0)
        val = lax.dot_general(xb[b:b + 1, :], mb,
                              (((1,), (1,)), ((), ())),
                              preferred_element_type=F32)
        encval_ref[b:b + 1, :] = jnp.where(jrow >= num[b:b + 1, :], PAD_ID, val)


def _prep(xb_pad):
    return pl.pallas_call(
        _prep_kernel,
        out_shape=[
            jax.ShapeDtypeStruct((B, T), F32),
            jax.ShapeDtypeStruct((B, T), F32),
            jax.ShapeDtypeStruct((B, 1), F32),
            jax.ShapeDtypeStruct((B, T), F32),
            jax.ShapeDtypeStruct((B, T), F32),
        ],
    )(xb_pad)


def _embed_kernel(v_ref, w1_ref, b1_ref, w2t_ref, b2_ref, emb_ref, em_ref,
                  ep_ref, out_ref):
    v = v_ref[...]
    h = v * w1_ref[...] + b1_ref[...]
    h = jnp.where(h >= 0.0, h, 0.1 * h)
    h2 = jnp.dot(h, w2t_ref[...], preferred_element_type=F32) + b2_ref[...]
    logits = h + h2
    logits = logits - jnp.max(logits, axis=1, keepdims=True)
    w = jnp.exp(logits)
    w = w / jnp.sum(w, axis=1, keepdims=True)
    e = jnp.dot(w.astype(BF16), emb_ref[...], preferred_element_type=F32)
    e = jnp.where(v == MASK_ID, em_ref[...], e)
    e = jnp.where(v == PAD_ID, ep_ref[...], e)
    out_ref[...] = e


def _embed(tokens, p):
    n = tokens.shape[0]
    bm = 528
    grid = (n // bm,)
    return pl.pallas_call(
        _embed_kernel,
        grid=grid,
        in_specs=[
            pl.BlockSpec((bm, 1), lambda i: (i, 0)),
            pl.BlockSpec((1, BIN), lambda i: (0, 0)),
            pl.BlockSpec((1, BIN), lambda i: (0, 0)),
            pl.BlockSpec((BIN, BIN), lambda i: (0, 0)),
            pl.BlockSpec((1, BIN), lambda i: (0, 0)),
            pl.BlockSpec((BIN, ED), lambda i: (0, 0)),
            pl.BlockSpec((1, ED), lambda i: (0, 0)),
            pl.BlockSpec((1, ED), lambda i: (0, 0)),
        ],
        out_specs=pl.BlockSpec((bm, ED), lambda i: (i, 0)),
        out_shape=jax.ShapeDtypeStruct((n, ED), F32),
    )(tokens,
      p["te_w1"].reshape(1, BIN),
      p["te_b1"].reshape(1, BIN),
      p["te_w2"].T,
      p["te_b2"].reshape(1, BIN),
      p["te_emb"].astype(BF16),
      p["te_emb_mask"].reshape(1, ED),
      p["te_emb_pad"].reshape(1, ED))


def _posgather_kernel(l_ref, r_ref, num_ref, h0e_ref, pe_ref, plast_ref,
                      out_ref):
    lb = l_ref[0]
    rb = r_ref[0]
    jcol = lax.broadcasted_iota(jnp.int32, (T, T), 0).astype(F32)
    mb = jnp.where((jcol == rb) & (lb > 0.0), 1.0, 0.0)
    pe = jnp.dot(mb.astype(BF16), pe_ref[...], preferred_element_type=F32)
    padcol = (lax.broadcasted_iota(jnp.int32, (T, 1), 0).astype(F32) >= num_ref[0]).astype(F32)
    pe = pe + padcol * plast_ref[...]
    out_ref[0] = h0e_ref[0] + pe


def _posgather(labels, rank, num, h0e, pos_pad, pos_last):
    return pl.pallas_call(
        _posgather_kernel,
        grid=(B,),
        in_specs=[
            pl.BlockSpec((1, 1, T), lambda b: (b, 0, 0)),
            pl.BlockSpec((1, 1, T), lambda b: (b, 0, 0)),
            pl.BlockSpec((1, 1, 1), lambda b: (b, 0, 0)),
            pl.BlockSpec((1, T, ED), lambda b: (b, 0, 0)),
            pl.BlockSpec((T, ED), lambda b: (0, 0)),
            pl.BlockSpec((1, ED), lambda b: (0, 0)),
        ],
        out_specs=pl.BlockSpec((1, T, ED), lambda b: (b, 0, 0)),
        out_shape=jax.ShapeDtypeStruct((B, T, ED), F32),
    )(labels.reshape(B, 1, T), rank.reshape(B, 1, T), num.reshape(B, 1, 1),
      h0e, pos_pad, pos_last)


def _stack_kernel(x_ref, wqkv_ref, bqkv_ref, wo_ref, bo_ref, g1_ref, bb1_ref,
                  w1_ref, b1_ref, w2_ref, b2_ref, g2_ref, bb2_ref, mask_ref,
                  out_ref, acc_ref, *, heads, dim, ff):
    x = x_ref[0]
    qkv = lax.dot_general(x.astype(BF16), wqkv_ref[0],
                          (((1,), (1,)), ((), ())),
                          preferred_element_type=F32) + bqkv_ref[0]
    dh = dim // heads
    scale = 1.0 / math.sqrt(dh)
    mask = mask_ref[0]
    for h in range(heads):
        q = qkv[:, h * dh:(h + 1) * dh]
        k = qkv[:, dim + h * dh:dim + (h + 1) * dh]
        v = qkv[:, 2 * dim + h * dh:2 * dim + (h + 1) * dh]
        s = lax.dot_general(q.astype(BF16), k.astype(BF16),
                            (((1,), (1,)), ((), ())),
                            preferred_element_type=F32) * scale + mask
        ps = jnp.exp(s)
        rs = 1.0 / jnp.sum(ps, axis=1, keepdims=True)
        acc_ref[:, h * dh:(h + 1) * dh] = rs * jnp.dot(
            ps.astype(BF16), v.astype(BF16), preferred_element_type=F32)
    o = lax.dot_general(acc_ref[...].astype(BF16), wo_ref[0],
                        (((1,), (1,)), ((), ())),
                        preferred_element_type=F32) + bo_ref[0]
    y = x + o
    m = jnp.mean(y, axis=1, keepdims=True)
    var = jnp.mean(y * y, axis=1, keepdims=True) - m * m
    y = (y - m) * lax.rsqrt(var + 1e-5) * g1_ref[0] + bb1_ref[0]

    hh = lax.dot_general(y.astype(BF16), w1_ref[0],
                         (((1,), (1,)), ((), ())),
                         preferred_element_type=F32) + b1_ref[0]
    hh = hh * 0.5 * (1.0 + lax.erf(hh * (1.0 / math.sqrt(2.0))))
    o2 = lax.dot_general(hh.astype(BF16), w2_ref[0],
                         (((1,), (1,)), ((), ())),
                         preferred_element_type=F32) + b2_ref[0]
    y2 = y + o2
    m2 = jnp.mean(y2, axis=1, keepdims=True)
    var2 = jnp.mean(y2 * y2, axis=1, keepdims=True) - m2 * m2
    out_ref[0] = (y2 - m2) * lax.rsqrt(var2 + 1e-5) * g2_ref[0] + bb2_ref[0]


def _stack(x, layers, mask, heads, dim, ff):
    import functools
    depth = len(layers)
    wqkv = jnp.stack([p["Wqkv"] for p in layers]).astype(BF16)
    bqkv = jnp.stack([p["bqkv"].reshape(1, 3 * dim) for p in layers])
    wo = jnp.stack([p["Wo"] for p in layers]).astype(BF16)
    bo = jnp.stack([p["bo"].reshape(1, dim) for p in layers])
    g1 = jnp.stack([p["ln1_g"].reshape(1, dim) for p in layers])
    bb1 = jnp.stack([p["ln1_b"].reshape(1, dim) for p in layers])
    w1 = jnp.stack([p["W1"] for p in layers]).astype(BF16)
    b1 = jnp.stack([p["b1"].reshape(1, ff) for p in layers])
    w2 = jnp.stack([p["W2"] for p in layers]).astype(BF16)
    b2 = jnp.stack([p["b2"].reshape(1, dim) for p in layers])
    g2 = jnp.stack([p["ln2_g"].reshape(1, dim) for p in layers])
    bb2 = jnp.stack([p["ln2_b"].reshape(1, dim) for p in layers])
    lb = lambda l, b: (b, 0, 0)
    lw3 = lambda l, b: (l, 0, 0)
    return pl.pallas_call(
        functools.partial(_stack_kernel, heads=heads, dim=dim, ff=ff),
        grid=(depth, B),
        in_specs=[
            pl.BlockSpec((1, T, dim), lb),
            pl.BlockSpec((1, 3 * dim, dim), lw3),
            pl.BlockSpec((1, 1, 3 * dim), lw3),
            pl.BlockSpec((1, dim, dim), lw3),
            pl.BlockSpec((1, 1, dim), lw3),
            pl.BlockSpec((1, 1, dim), lw3),
            pl.BlockSpec((1, 1, dim), lw3),
            pl.BlockSpec((1, ff, dim), lw3),
            pl.BlockSpec((1, 1, ff), lw3),
            pl.BlockSpec((1, dim, ff), lw3),
            pl.BlockSpec((1, 1, dim), lw3),
            pl.BlockSpec((1, 1, dim), lw3),
            pl.BlockSpec((1, 1, dim), lw3),
            pl.BlockSpec((1, 1, T), lambda l, b: (b, 0, 0)),
        ],
        out_specs=pl.BlockSpec((1, T, dim), lb),
        out_shape=jax.ShapeDtypeStruct((B, T, dim), F32),
        scratch_shapes=[pltpu.VMEM((T, dim), F32)],
        input_output_aliases={0: 0},
    )(x, wqkv, bqkv, wo, bo, g1, bb1, w1, b1, w2, b2, g2, bb2,
      mask.reshape(B, 1, T))


def _ln_kernel(x_ref, g_ref, bb_ref, out_ref):
    x = x_ref[0]
    m = jnp.mean(x, axis=1, keepdims=True)
    d = x - m
    var = jnp.mean(d * d, axis=1, keepdims=True)
    out_ref[0] = d * lax.rsqrt(var + 1e-5) * g_ref[...] + bb_ref[...]


def _ln(x, g, bb, dim):
    return pl.pallas_call(
        _ln_kernel,
        grid=(B,),
        in_specs=[
            pl.BlockSpec((1, T, dim), lambda b: (b, 0, 0)),
            pl.BlockSpec((1, dim), lambda b: (0, 0)),
            pl.BlockSpec((1, dim), lambda b: (0, 0)),
        ],
        out_specs=pl.BlockSpec((1, T, dim), lambda b: (b, 0, 0)),
        out_shape=jax.ShapeDtypeStruct((B, T, dim), F32),
    )(x, g.reshape(1, dim), bb.reshape(1, dim))


def _scatter_kernel(l_ref, r_ref, henc_ref, e_ref, pos_ref, wdec_ref,
                    bdec_ref, out_ref):
    lb = l_ref[0]
    rb = r_ref[0]
    jcol = lax.broadcasted_iota(jnp.int32, (T, T), 0).astype(F32)
    mb = jnp.where((jcol == rb) & (lb > 0.0), 1.0, 0.0)
    gathered = lax.dot_general(mb.astype(BF16), henc_ref[0].astype(BF16),
                               (((0,), (0,)), ((), ())),
                               preferred_element_type=F32)
    ones = jnp.ones((T, 1), F32)
    lcol = lax.dot_general(mb, ones, (((0,), (0,)), ((), ())),
                           preferred_element_type=F32)
    base = gathered + (1.0 - lcol) * e_ref[0] + pos_ref[...]
    out_ref[0] = jnp.dot(base.astype(BF16), wdec_ref[...],
                         preferred_element_type=F32) + bdec_ref[...]


def _scatterback(labels, rank, h_enc, e_full, dec_pos_pad, wdec_t, bdec):
    return pl.pallas_call(
        _scatter_kernel,
        grid=(B,),
        in_specs=[
            pl.BlockSpec((1, 1, T), lambda b: (b, 0, 0)),
            pl.BlockSpec((1, 1, T), lambda b: (b, 0, 0)),
            pl.BlockSpec((1, T, ED), lambda b: (b, 0, 0)),
            pl.BlockSpec((1, T, ED), lambda b: (b, 0, 0)),
            pl.BlockSpec((T, ED), lambda b: (0, 0)),
            pl.BlockSpec((ED, DD), lambda b: (0, 0)),
            pl.BlockSpec((1, DD), lambda b: (0, 0)),
        ],
        out_specs=pl.BlockSpec((1, T, DD), lambda b: (b, 0, 0)),
        out_shape=jax.ShapeDtypeStruct((B, T, DD), F32),
    )(labels.reshape(B, 1, T), rank.reshape(B, 1, T), h_enc, e_full,
      dec_pos_pad, wdec_t, bdec)


def kernel(x, params):
    genes = x[:, :-1]
    li = jnp.log10(x[:, -1:])
    xb = jnp.concatenate([genes, li, li], axis=1)
    xb_pad = jnp.pad(xb, ((0, 0), (0, T - S)))

    labels, rank, num, maskadd_enc, enc_val = _prep(xb_pad)

    tokens = jnp.concatenate(
        [enc_val.reshape(-1, 1), xb_pad.reshape(-1, 1)], axis=0)
    e_all = _embed(tokens, params)
    h0e = e_all[:B * T].reshape(B, T, ED)
    e_full = e_all[B * T:].reshape(B, T, ED)

    pos_pad = jnp.pad(params["pos_emb"][:S], ((0, T - S), (0, 0))).astype(BF16)
    pos_last = params["pos_emb"][S:S + 1]
    h = _posgather(labels, rank, num, h0e, pos_pad, pos_last)

    h = _stack(h, params["enc_layers"], maskadd_enc, EH, ED, 4 * ED)
    h = _ln(h, params["enc_norm_g"], params["enc_norm_b"], ED)

    dec_pos_pad = jnp.pad(params["dec_pos_emb"][:S], ((0, T - S), (0, 0)))
    d = _scatterback(labels, rank, h, e_full, dec_pos_pad,
                     params["dec_embed_W"].T.astype(BF16),
                     params["dec_embed_b"].reshape(1, DD))

    maskadd_dec = jnp.broadcast_to(
        jnp.where(jnp.arange(T, dtype=F32) >= S, NEG, 0.0)[None, :], (B, T))

    d = _stack(d, params["dec_layers"], maskadd_dec, DH, DD, 4 * DD)
    d = _ln(d, params["dec_norm_g"], params["dec_norm_b"], DD)
    return d[:, :S, :]
